# baseline (speedup 1.0000x reference)
import functools

import jax
import jax.numpy as jnp
from jax import lax
from jax.experimental import pallas as pl
from jax.experimental.pallas import tpu as pltpu
from jax.experimental.pallas import tpu_sc as plsc

B = 32
H = 384
W = 384
HIDDEN = 128
KC = 4608
NSTEPS = (H * W) // KC
WPACK = W // 2
IMG_WORDS = H * WPACK
ROWS_PER_CHUNK = 8


def _locnet_body(x_ref, w1_ref, b1_ref, w2_ref, b2_ref, th_ref, acc_ref):
    s = pl.program_id(0)

    @pl.when(s == 0)
    def _():
        acc_ref[...] = jnp.zeros_like(acc_ref)

    acc_ref[...] += lax.dot_general(
        x_ref[...].astype(jnp.bfloat16), w1_ref[...].astype(jnp.bfloat16),
        (((1,), (0,)), ((), ())),
        preferred_element_type=jnp.float32)

    @pl.when(s == NSTEPS - 1)
    def _():
        h = jnp.maximum(acc_ref[...] + b1_ref[...], 0.0)
        th = lax.dot_general(
            h.astype(jnp.bfloat16), w2_ref[...].astype(jnp.bfloat16),
            (((1,), (0,)), ((), ())),
            preferred_element_type=jnp.float32) + b2_ref[...]
        th_ref[...] = th.astype(jnp.bfloat16).astype(jnp.float32)


@jax.jit
def _locnet(xf, w1, b1, w2p, b2p):
    return pl.pallas_call(
        _locnet_body,
        grid=(NSTEPS,),
        in_specs=[
            pl.BlockSpec((B, KC), lambda s: (0, s)),
            pl.BlockSpec((KC, HIDDEN), lambda s: (s, 0)),
            pl.BlockSpec((1, HIDDEN), lambda s: (0, 0)),
            pl.BlockSpec((HIDDEN, 16), lambda s: (0, 0)),
            pl.BlockSpec((1, 16), lambda s: (0, 0)),
        ],
        out_specs=pl.BlockSpec((B, 16), lambda s: (0, 0)),
        out_shape=jax.ShapeDtypeStruct((B, 16), jnp.float32),
        scratch_shapes=[pltpu.VMEM((B, HIDDEN), jnp.float32)],
    )(xf, w1, b1, w2p, b2p)


F32_CHUNK = 8 * W
N_F32_CHUNKS = (H * W) // F32_CHUNK


def _pack_body(ximg_hbm, pk_hbm, img_v, fb0, fb1, sem0, sem1, osem):
    c = lax.axis_index("c")
    s = lax.axis_index("s")
    b = s * 2 + c

    lane = lax.iota(jnp.int32, 16)
    lane2 = lane * 2

    fbufs = (fb0, fb1)
    sems = (sem0, sem1)
    descs = [None, None]
    odescs = [None] * N_F32_CHUNKS

    def start(cc):
        descs[cc % 2] = pltpu.async_copy(
            ximg_hbm.at[b, pl.ds(cc * F32_CHUNK, F32_CHUNK)],
            fbufs[cc % 2], sems[cc % 2])

    def pack_chunk(cc):
        fb = fbufs[cc % 2]

        @plsc.parallel_loop(0, F32_CHUNK // 32, unroll=4)
        def pk(k):
            base = k * 32
            a = plsc.load_gather(fb, [base + lane2])
            bb = plsc.load_gather(fb, [base + lane2 + 1])
            w = plsc.bitcast(
                plsc.pack(a, bb, format=plsc.PackFormat.INTERLEAVED), jnp.int32)
            img_v[pl.ds(cc * (F32_CHUNK // 2) + k * 16, 16)] = w

    start(0)
    for cc in range(N_F32_CHUNKS):
        if cc + 1 < N_F32_CHUNKS:
            start(cc + 1)
        descs[cc % 2].wait()
        pack_chunk(cc)
        odescs[cc] = pltpu.async_copy(
            img_v.at[pl.ds(cc * (F32_CHUNK // 2), F32_CHUNK // 2)],
            pk_hbm.at[b, pl.ds(cc * (F32_CHUNK // 2), F32_CHUNK // 2)], osem)
    for cc in range(N_F32_CHUNKS):
        odescs[cc].wait()


@jax.jit
def _pack(xf):
    mesh = plsc.VectorSubcoreMesh(
        core_axis_name="c", subcore_axis_name="s", num_cores=2, num_subcores=16)
    return pl.kernel(
        _pack_body,
        out_type=jax.ShapeDtypeStruct((B, IMG_WORDS), jnp.int32),
        mesh=mesh,
        compiler_params=pltpu.CompilerParams(needs_layout_passes=False),
        scratch_types=[
            pltpu.VMEM((IMG_WORDS,), jnp.int32),
            pltpu.VMEM((F32_CHUNK,), jnp.float32),
            pltpu.VMEM((F32_CHUNK,), jnp.float32),
            pltpu.SemaphoreType.DMA,
            pltpu.SemaphoreType.DMA,
            pltpu.SemaphoreType.DMA,
        ],
    )(xf)


def _warp_body(pk_hbm, th_hbm, out_hbm, img_v, th_v, jtab, ob0, isem):
    c = lax.axis_index("c")
    s = lax.axis_index("s")
    b = s * 2 + c

    idesc = pltpu.async_copy(pk_hbm.at[b], img_v, isem)
    pltpu.sync_copy(th_hbm.at[b], th_v)

    lane = lax.iota(jnp.int32, 16)
    th = th_v[...]

    def bcast(k):
        sel = jnp.where(lane == k, th, jnp.zeros((16,), jnp.float32))
        return jnp.full((16,), jnp.sum(sel, axis=0))

    t00 = bcast(0)
    t01 = bcast(1)
    t02 = bcast(2)
    t10 = bcast(3)
    t11 = bcast(4)
    t12 = bcast(5)

    lanei = lax.iota(jnp.int32, 16)
    lanef = lanei.astype(jnp.float32)
    xmax = jnp.full((16,), W - 1, jnp.float32)
    ymax = jnp.full((16,), H - 1, jnp.float32)
    zero = jnp.zeros((16,), jnp.float32)

    def rbf(v):
        bits = plsc.bitcast(v, jnp.int32)
        r = (bits + 0x7FFF + ((bits >> 16) & 1)) & jnp.int32(-65536)
        return plsc.bitcast(r, jnp.float32)

    @plsc.parallel_loop(0, W // 16, unroll=2)
    def fill_jtab(g):
        jtab[pl.ds(g * 16, 16)] = rbf(
            jnp.full((16,), g * 16, jnp.int32).astype(jnp.float32) + lanef)

    idesc.wait()

    def do_row(i, rr, obuf):
        iv = rbf(jnp.full((16,), i, jnp.int32).astype(jnp.float32))
        xrow = t01 * iv + t02
        yrow = t11 * iv + t12

        @plsc.parallel_loop(0, W // 16, unroll=6)
        def group(g):
            jv = jtab[pl.ds(g * 16, 16)]
            x = t00 * jv + xrow
            y = t10 * jv + yrow
            xlo = jnp.minimum(jnp.maximum(x, zero), xmax)
            xhi = jnp.minimum(jnp.maximum(x + 1.0, zero), xmax)
            ylo = jnp.minimum(jnp.maximum(y, zero), ymax)
            yhi = jnp.minimum(jnp.maximum(y + 1.0, zero), ymax)
            x0 = xlo.astype(jnp.int32)
            x1 = xhi.astype(jnp.int32)
            y0 = ylo.astype(jnp.int32)
            y1 = yhi.astype(jnp.int32)
            x0f = x0.astype(jnp.float32)
            x1f = x1.astype(jnp.float32)
            y0f = y0.astype(jnp.float32)
            y1f = y1.astype(jnp.float32)
            r0 = y0 * WPACK
            r1 = y1 * WPACK
            wxa = x0 >> 1
            wxc = x1 >> 1
            sha = (x0 & 1) << 4
            shc = (x1 & 1) << 4

            def tap(widx, sh):
                wv = plsc.load_gather(img_v, [widx])
                return plsc.bitcast((wv >> sh) << 16, jnp.float32)

            pa = tap(r0 + wxa, sha)
            pc = tap(r0 + wxc, shc)
            pb = tap(r1 + wxa, sha)
            pd = tap(r1 + wxc, shc)

            res = ((y1f - y) * ((x1f - x) * pa + (x - x0f) * pc)
                   + (y - y0f) * ((x1f - x) * pb + (x - x0f) * pd))
            obuf[rr, pl.ds(g * 16, 16)] = res

    def chunk_body(ch, _):
        for rr in range(ROWS_PER_CHUNK):
            do_row(ch * ROWS_PER_CHUNK + rr, rr, ob0)
        pltpu.sync_copy(ob0, out_hbm.at[b, pl.ds(ch * ROWS_PER_CHUNK, ROWS_PER_CHUNK)])
        return 0

    lax.fori_loop(0, H // ROWS_PER_CHUNK, chunk_body, 0)


@jax.jit
def _warp(img, th):
    mesh = plsc.VectorSubcoreMesh(
        core_axis_name="c", subcore_axis_name="s", num_cores=2, num_subcores=16)
    return pl.kernel(
        _warp_body,
        out_type=jax.ShapeDtypeStruct((B, H, W), jnp.float32),
        mesh=mesh,
        compiler_params=pltpu.CompilerParams(needs_layout_passes=False),
        scratch_types=[
            pltpu.VMEM((IMG_WORDS,), jnp.int32),
            pltpu.VMEM((16,), jnp.float32),
            pltpu.VMEM((W,), jnp.float32),
            pltpu.VMEM((ROWS_PER_CHUNK, W), jnp.float32),
            pltpu.SemaphoreType.DMA,
        ],
    )(img, th)


def kernel(X, W1, b1, W2, b2):
    xf = X.reshape(B, H * W)
    w2p = jnp.zeros((HIDDEN, 16), jnp.float32).at[:, :6].set(W2)
    b2p = jnp.zeros((16,), jnp.float32).at[:6].set(b2)
    packed = _pack(xf)
    theta16 = _locnet(xf, W1, b1.reshape(1, HIDDEN), w2p, b2p.reshape(1, 16))
    return _warp(packed, theta16)


# speedup vs baseline: 1.3003x; 1.3003x over previous
import functools

import jax
import jax.numpy as jnp
from jax import lax
from jax.experimental import pallas as pl
from jax.experimental.pallas import tpu as pltpu
from jax.experimental.pallas import tpu_sc as plsc

B = 32
H = 384
W = 384
HIDDEN = 128
KC = 4608
NSTEPS = (H * W) // KC
WPACK = W // 2
IMG_WORDS = H * WPACK
ROWS_PER_CHUNK = 8


def _locnet_body(x_ref, w1_ref, b1_ref, w2_ref, b2_ref, th_ref, acc_ref):
    s = pl.program_id(0)

    @pl.when(s == 0)
    def _():
        acc_ref[...] = jnp.zeros_like(acc_ref)

    acc_ref[...] += lax.dot_general(
        x_ref[...].astype(jnp.bfloat16), w1_ref[...].astype(jnp.bfloat16),
        (((1,), (0,)), ((), ())),
        preferred_element_type=jnp.float32)

    @pl.when(s == NSTEPS - 1)
    def _():
        h = jnp.maximum(acc_ref[...] + b1_ref[...], 0.0)
        th = lax.dot_general(
            h.astype(jnp.bfloat16), w2_ref[...].astype(jnp.bfloat16),
            (((1,), (0,)), ((), ())),
            preferred_element_type=jnp.float32) + b2_ref[...]
        th_ref[...] = th.astype(jnp.bfloat16).astype(jnp.float32)


@jax.jit
def _locnet(xf, w1, b1, w2p, b2p):
    return pl.pallas_call(
        _locnet_body,
        grid=(NSTEPS,),
        in_specs=[
            pl.BlockSpec((B, KC), lambda s: (0, s)),
            pl.BlockSpec((KC, HIDDEN), lambda s: (s, 0)),
            pl.BlockSpec((1, HIDDEN), lambda s: (0, 0)),
            pl.BlockSpec((HIDDEN, 16), lambda s: (0, 0)),
            pl.BlockSpec((1, 16), lambda s: (0, 0)),
        ],
        out_specs=pl.BlockSpec((B, 16), lambda s: (0, 0)),
        out_shape=jax.ShapeDtypeStruct((B, 16), jnp.float32),
        scratch_shapes=[pltpu.VMEM((B, HIDDEN), jnp.float32)],
    )(xf, w1, b1, w2p, b2p)


F32_CHUNK = 8 * W
N_F32_CHUNKS = (H * W) // F32_CHUNK


def _pack_body(ximg_hbm, pk_hbm, img_v, fb0, fb1, sem0, sem1, osem):
    c = lax.axis_index("c")
    s = lax.axis_index("s")
    b = s * 2 + c

    lane = lax.iota(jnp.int32, 16)
    lane2 = lane * 2

    fbufs = (fb0, fb1)
    sems = (sem0, sem1)
    descs = [None, None]
    odescs = [None] * N_F32_CHUNKS

    def start(cc):
        descs[cc % 2] = pltpu.async_copy(
            ximg_hbm.at[b, pl.ds(cc * F32_CHUNK, F32_CHUNK)],
            fbufs[cc % 2], sems[cc % 2])

    def pack_chunk(cc):
        fb = fbufs[cc % 2]

        @plsc.parallel_loop(0, F32_CHUNK // 32, unroll=4)
        def pk(k):
            base = k * 32
            a = plsc.load_gather(fb, [base + lane2])
            bb = plsc.load_gather(fb, [base + lane2 + 1])
            w = plsc.bitcast(
                plsc.pack(a, bb, format=plsc.PackFormat.INTERLEAVED), jnp.int32)
            img_v[pl.ds(cc * (F32_CHUNK // 2) + k * 16, 16)] = w

    start(0)
    for cc in range(N_F32_CHUNKS):
        if cc + 1 < N_F32_CHUNKS:
            start(cc + 1)
        descs[cc % 2].wait()
        pack_chunk(cc)
        odescs[cc] = pltpu.async_copy(
            img_v.at[pl.ds(cc * (F32_CHUNK // 2), F32_CHUNK // 2)],
            pk_hbm.at[b, pl.ds(cc * (F32_CHUNK // 2), F32_CHUNK // 2)], osem)
    for cc in range(N_F32_CHUNKS):
        odescs[cc].wait()


@jax.jit
def _pack(xf):
    mesh = plsc.VectorSubcoreMesh(
        core_axis_name="c", subcore_axis_name="s", num_cores=2, num_subcores=16)
    return pl.kernel(
        _pack_body,
        out_type=jax.ShapeDtypeStruct((B, IMG_WORDS), jnp.int32),
        mesh=mesh,
        compiler_params=pltpu.CompilerParams(needs_layout_passes=False),
        scratch_types=[
            pltpu.VMEM((IMG_WORDS,), jnp.int32),
            pltpu.VMEM((F32_CHUNK,), jnp.float32),
            pltpu.VMEM((F32_CHUNK,), jnp.float32),
            pltpu.SemaphoreType.DMA,
            pltpu.SemaphoreType.DMA,
            pltpu.SemaphoreType.DMA,
        ],
    )(xf)


def _warp_body(pk_hbm, th_hbm, out_hbm, img_v, th_v, jtab, ob0, isem):
    c = lax.axis_index("c")
    s = lax.axis_index("s")
    b = s * 2 + c

    idesc = pltpu.async_copy(pk_hbm.at[b], img_v, isem)
    pltpu.sync_copy(th_hbm.at[b], th_v)

    lane = lax.iota(jnp.int32, 16)
    th = th_v[...]

    def bcast(k):
        sel = jnp.where(lane == k, th, jnp.zeros((16,), jnp.float32))
        return jnp.full((16,), jnp.sum(sel, axis=0))

    t00 = bcast(0)
    t01 = bcast(1)
    t02 = bcast(2)
    t10 = bcast(3)
    t11 = bcast(4)
    t12 = bcast(5)

    lanei = lax.iota(jnp.int32, 16)
    lanef = lanei.astype(jnp.float32)
    xmax = jnp.full((16,), W - 1, jnp.float32)
    ymax = jnp.full((16,), H - 1, jnp.float32)
    zero = jnp.zeros((16,), jnp.float32)

    def rbf(v):
        bits = plsc.bitcast(v, jnp.int32)
        r = (bits + 0x7FFF + ((bits >> 16) & 1)) & jnp.int32(-65536)
        return plsc.bitcast(r, jnp.float32)

    @plsc.parallel_loop(0, W // 16, unroll=2)
    def fill_jtab(g):
        jtab[pl.ds(g * 16, 16)] = rbf(
            jnp.full((16,), g * 16, jnp.int32).astype(jnp.float32) + lanef)

    idesc.wait()

    def do_row(i, rr, obuf):
        iv = rbf(jnp.full((16,), i, jnp.int32).astype(jnp.float32))
        xrow = t01 * iv + t02
        yrow = t11 * iv + t12

        @plsc.parallel_loop(0, W // 16, unroll=3)
        def group(g):
            jv = jtab[pl.ds(g * 16, 16)]
            x = t00 * jv + xrow
            y = t10 * jv + yrow
            xlo = jnp.minimum(jnp.maximum(x, zero), xmax)
            xhi = jnp.minimum(jnp.maximum(x + 1.0, zero), xmax)
            ylo = jnp.minimum(jnp.maximum(y, zero), ymax)
            yhi = jnp.minimum(jnp.maximum(y + 1.0, zero), ymax)
            x0 = xlo.astype(jnp.int32)
            x1 = xhi.astype(jnp.int32)
            y0 = ylo.astype(jnp.int32)
            y1 = yhi.astype(jnp.int32)
            x0f = x0.astype(jnp.float32)
            x1f = x1.astype(jnp.float32)
            y0f = y0.astype(jnp.float32)
            y1f = y1.astype(jnp.float32)
            r0 = y0 * WPACK
            r1 = y1 * WPACK
            wxa = x0 >> 1
            wxc = x1 >> 1
            sha = (x0 & 1) << 4
            shc = (x1 & 1) << 4

            def tap(widx, sh):
                wv = plsc.load_gather(img_v, [widx])
                return plsc.bitcast((wv >> sh) << 16, jnp.float32)

            pa = tap(r0 + wxa, sha)
            pc = tap(r0 + wxc, shc)
            pb = tap(r1 + wxa, sha)
            pd = tap(r1 + wxc, shc)

            res = ((y1f - y) * ((x1f - x) * pa + (x - x0f) * pc)
                   + (y - y0f) * ((x1f - x) * pb + (x - x0f) * pd))
            obuf[rr, pl.ds(g * 16, 16)] = res

    def chunk_body(ch, _):
        for rr in range(ROWS_PER_CHUNK):
            do_row(ch * ROWS_PER_CHUNK + rr, rr, ob0)
        pltpu.sync_copy(ob0, out_hbm.at[b, pl.ds(ch * ROWS_PER_CHUNK, ROWS_PER_CHUNK)])
        return 0

    lax.fori_loop(0, H // ROWS_PER_CHUNK, chunk_body, 0)


@jax.jit
def _warp(img, th):
    mesh = plsc.VectorSubcoreMesh(
        core_axis_name="c", subcore_axis_name="s", num_cores=2, num_subcores=16)
    return pl.kernel(
        _warp_body,
        out_type=jax.ShapeDtypeStruct((B, H, W), jnp.float32),
        mesh=mesh,
        compiler_params=pltpu.CompilerParams(needs_layout_passes=False),
        scratch_types=[
            pltpu.VMEM((IMG_WORDS,), jnp.int32),
            pltpu.VMEM((16,), jnp.float32),
            pltpu.VMEM((W,), jnp.float32),
            pltpu.VMEM((ROWS_PER_CHUNK, W), jnp.float32),
            pltpu.SemaphoreType.DMA,
        ],
    )(img, th)


def kernel(X, W1, b1, W2, b2):
    xf = X.reshape(B, H * W)
    w2p = jnp.zeros((HIDDEN, 16), jnp.float32).at[:, :6].set(W2)
    b2p = jnp.zeros((16,), jnp.float32).at[:6].set(b2)
    packed = _pack(xf)
    theta16 = _locnet(xf, W1, b1.reshape(1, HIDDEN), w2p, b2p.reshape(1, 16))
    return _warp(packed, theta16)


# speedup vs baseline: 1.3594x; 1.0455x over previous
import functools

import jax
import jax.numpy as jnp
from jax import lax
from jax.experimental import pallas as pl
from jax.experimental.pallas import tpu as pltpu
from jax.experimental.pallas import tpu_sc as plsc

B = 32
H = 384
W = 384
HIDDEN = 128
KC = 9216
NSTEPS = (H * W) // KC
WPACK = W // 2
IMG_WORDS = H * WPACK
ROWS_PER_CHUNK = 8


def _locnet_body(x_ref, w1_ref, b1_ref, w2_ref, b2_ref, th_ref, acc_ref):
    s = pl.program_id(0)

    @pl.when(s == 0)
    def _():
        acc_ref[...] = jnp.zeros_like(acc_ref)

    acc_ref[...] += lax.dot_general(
        x_ref[...].astype(jnp.bfloat16), w1_ref[...].astype(jnp.bfloat16),
        (((1,), (0,)), ((), ())),
        preferred_element_type=jnp.float32)

    @pl.when(s == NSTEPS - 1)
    def _():
        h = jnp.maximum(acc_ref[...] + b1_ref[...], 0.0)
        th = lax.dot_general(
            h.astype(jnp.bfloat16), w2_ref[...].astype(jnp.bfloat16),
            (((1,), (0,)), ((), ())),
            preferred_element_type=jnp.float32) + b2_ref[...]
        th_ref[...] = th.astype(jnp.bfloat16).astype(jnp.float32)


@jax.jit
def _locnet(xf, w1, b1, w2p, b2p):
    return pl.pallas_call(
        _locnet_body,
        grid=(NSTEPS,),
        in_specs=[
            pl.BlockSpec((B, KC), lambda s: (0, s)),
            pl.BlockSpec((KC, HIDDEN), lambda s: (s, 0)),
            pl.BlockSpec((1, HIDDEN), lambda s: (0, 0)),
            pl.BlockSpec((HIDDEN, 16), lambda s: (0, 0)),
            pl.BlockSpec((1, 16), lambda s: (0, 0)),
        ],
        out_specs=pl.BlockSpec((B, 16), lambda s: (0, 0)),
        out_shape=jax.ShapeDtypeStruct((B, 16), jnp.float32),
        scratch_shapes=[pltpu.VMEM((B, HIDDEN), jnp.float32)],
    )(xf, w1, b1, w2p, b2p)


F32_CHUNK = 8 * W
N_F32_CHUNKS = (H * W) // F32_CHUNK


def _pack_body(ximg_hbm, pk_hbm, img_v, fb0, fb1, sem0, sem1, osem):
    c = lax.axis_index("c")
    s = lax.axis_index("s")
    b = s * 2 + c

    lane = lax.iota(jnp.int32, 16)
    lane2 = lane * 2

    fbufs = (fb0, fb1)
    sems = (sem0, sem1)
    descs = [None, None]
    odescs = [None] * N_F32_CHUNKS

    def start(cc):
        descs[cc % 2] = pltpu.async_copy(
            ximg_hbm.at[b, pl.ds(cc * F32_CHUNK, F32_CHUNK)],
            fbufs[cc % 2], sems[cc % 2])

    def pack_chunk(cc):
        fb = fbufs[cc % 2]

        @plsc.parallel_loop(0, F32_CHUNK // 32, unroll=4)
        def pk(k):
            base = k * 32
            a = plsc.load_gather(fb, [base + lane2])
            bb = plsc.load_gather(fb, [base + lane2 + 1])
            w = plsc.bitcast(
                plsc.pack(a, bb, format=plsc.PackFormat.INTERLEAVED), jnp.int32)
            img_v[pl.ds(cc * (F32_CHUNK // 2) + k * 16, 16)] = w

    start(0)
    for cc in range(N_F32_CHUNKS):
        if cc + 1 < N_F32_CHUNKS:
            start(cc + 1)
        descs[cc % 2].wait()
        pack_chunk(cc)
        odescs[cc] = pltpu.async_copy(
            img_v.at[pl.ds(cc * (F32_CHUNK // 2), F32_CHUNK // 2)],
            pk_hbm.at[b, pl.ds(cc * (F32_CHUNK // 2), F32_CHUNK // 2)], osem)
    for cc in range(N_F32_CHUNKS):
        odescs[cc].wait()


@jax.jit
def _pack(xf):
    mesh = plsc.VectorSubcoreMesh(
        core_axis_name="c", subcore_axis_name="s", num_cores=2, num_subcores=16)
    return pl.kernel(
        _pack_body,
        out_type=jax.ShapeDtypeStruct((B, IMG_WORDS), jnp.int32),
        mesh=mesh,
        compiler_params=pltpu.CompilerParams(needs_layout_passes=False),
        scratch_types=[
            pltpu.VMEM((IMG_WORDS,), jnp.int32),
            pltpu.VMEM((F32_CHUNK,), jnp.float32),
            pltpu.VMEM((F32_CHUNK,), jnp.float32),
            pltpu.SemaphoreType.DMA,
            pltpu.SemaphoreType.DMA,
            pltpu.SemaphoreType.DMA,
        ],
    )(xf)


def _warp_body(pk_hbm, th_hbm, out_hbm, img_v, th_v, jtab, ob0, isem):
    c = lax.axis_index("c")
    s = lax.axis_index("s")
    b = s * 2 + c

    idesc = pltpu.async_copy(pk_hbm.at[b], img_v, isem)
    pltpu.sync_copy(th_hbm.at[b], th_v)

    lane = lax.iota(jnp.int32, 16)
    th = th_v[...]

    def bcast(k):
        sel = jnp.where(lane == k, th, jnp.zeros((16,), jnp.float32))
        return jnp.full((16,), jnp.sum(sel, axis=0))

    t00 = bcast(0)
    t01 = bcast(1)
    t02 = bcast(2)
    t10 = bcast(3)
    t11 = bcast(4)
    t12 = bcast(5)

    lanei = lax.iota(jnp.int32, 16)
    lanef = lanei.astype(jnp.float32)
    xmax = jnp.full((16,), W - 1, jnp.float32)
    ymax = jnp.full((16,), H - 1, jnp.float32)
    zero = jnp.zeros((16,), jnp.float32)

    def rbf(v):
        bits = plsc.bitcast(v, jnp.int32)
        r = (bits + 0x7FFF + ((bits >> 16) & 1)) & jnp.int32(-65536)
        return plsc.bitcast(r, jnp.float32)

    @plsc.parallel_loop(0, W // 16, unroll=2)
    def fill_jtab(g):
        jtab[pl.ds(g * 16, 16)] = rbf(
            jnp.full((16,), g * 16, jnp.int32).astype(jnp.float32) + lanef)

    idesc.wait()

    def do_row(i, rr, obuf):
        iv = rbf(jnp.full((16,), i, jnp.int32).astype(jnp.float32))
        xrow = t01 * iv + t02
        yrow = t11 * iv + t12

        @plsc.parallel_loop(0, W // 16, unroll=4)
        def group(g):
            jv = jtab[pl.ds(g * 16, 16)]
            x = t00 * jv + xrow
            y = t10 * jv + yrow
            xlo = jnp.minimum(jnp.maximum(x, zero), xmax)
            xhi = jnp.minimum(jnp.maximum(x + 1.0, zero), xmax)
            ylo = jnp.minimum(jnp.maximum(y, zero), ymax)
            yhi = jnp.minimum(jnp.maximum(y + 1.0, zero), ymax)
            x0 = xlo.astype(jnp.int32)
            x1 = xhi.astype(jnp.int32)
            y0 = ylo.astype(jnp.int32)
            y1 = yhi.astype(jnp.int32)
            x0f = x0.astype(jnp.float32)
            x1f = x1.astype(jnp.float32)
            y0f = y0.astype(jnp.float32)
            y1f = y1.astype(jnp.float32)
            r0 = y0 * WPACK
            r1 = y1 * WPACK
            wxa = x0 >> 1
            wxc = x1 >> 1
            sha = (x0 & 1) << 4
            shc = (x1 & 1) << 4

            def tap(widx, sh):
                wv = plsc.load_gather(img_v, [widx])
                return plsc.bitcast((wv >> sh) << 16, jnp.float32)

            pa = tap(r0 + wxa, sha)
            pc = tap(r0 + wxc, shc)
            pb = tap(r1 + wxa, sha)
            pd = tap(r1 + wxc, shc)

            res = ((y1f - y) * ((x1f - x) * pa + (x - x0f) * pc)
                   + (y - y0f) * ((x1f - x) * pb + (x - x0f) * pd))
            obuf[rr, pl.ds(g * 16, 16)] = res

    def chunk_body(ch, _):
        for rr in range(ROWS_PER_CHUNK):
            do_row(ch * ROWS_PER_CHUNK + rr, rr, ob0)
        pltpu.sync_copy(ob0, out_hbm.at[b, pl.ds(ch * ROWS_PER_CHUNK, ROWS_PER_CHUNK)])
        return 0

    lax.fori_loop(0, H // ROWS_PER_CHUNK, chunk_body, 0)


@jax.jit
def _warp(img, th):
    mesh = plsc.VectorSubcoreMesh(
        core_axis_name="c", subcore_axis_name="s", num_cores=2, num_subcores=16)
    return pl.kernel(
        _warp_body,
        out_type=jax.ShapeDtypeStruct((B, H, W), jnp.float32),
        mesh=mesh,
        compiler_params=pltpu.CompilerParams(needs_layout_passes=False),
        scratch_types=[
            pltpu.VMEM((IMG_WORDS,), jnp.int32),
            pltpu.VMEM((16,), jnp.float32),
            pltpu.VMEM((W,), jnp.float32),
            pltpu.VMEM((ROWS_PER_CHUNK, W), jnp.float32),
            pltpu.SemaphoreType.DMA,
        ],
    )(img, th)


def kernel(X, W1, b1, W2, b2):
    xf = X.reshape(B, H * W)
    w2p = jnp.zeros((HIDDEN, 16), jnp.float32).at[:, :6].set(W2)
    b2p = jnp.zeros((16,), jnp.float32).at[:6].set(b2)
    packed = _pack(xf)
    theta16 = _locnet(xf, W1, b1.reshape(1, HIDDEN), w2p, b2p.reshape(1, 16))
    return _warp(packed, theta16)


# speedup vs baseline: 1.3930x; 1.0247x over previous
import functools

import jax
import jax.numpy as jnp
from jax import lax
from jax.experimental import pallas as pl
from jax.experimental.pallas import tpu as pltpu
from jax.experimental.pallas import tpu_sc as plsc

B = 32
H = 384
W = 384
HIDDEN = 128
KC = 4608
NSTEPS = (H * W) // KC
WPACK = W // 2
IMG_WORDS = H * WPACK
ROWS_PER_CHUNK = 8


def _locnet_body(x_ref, w1_ref, b1_ref, w2_ref, b2_ref, th_ref, acc_ref):
    s = pl.program_id(0)

    @pl.when(s == 0)
    def _():
        acc_ref[...] = jnp.zeros_like(acc_ref)

    acc_ref[...] += lax.dot_general(
        x_ref[...].astype(jnp.bfloat16), w1_ref[...].astype(jnp.bfloat16),
        (((1,), (0,)), ((), ())),
        preferred_element_type=jnp.float32)

    @pl.when(s == NSTEPS - 1)
    def _():
        h = jnp.maximum(acc_ref[...] + b1_ref[...], 0.0)
        th = lax.dot_general(
            h.astype(jnp.bfloat16), w2_ref[...].astype(jnp.bfloat16),
            (((1,), (0,)), ((), ())),
            preferred_element_type=jnp.float32) + b2_ref[...]
        th_ref[...] = th.astype(jnp.bfloat16).astype(jnp.float32)


@jax.jit
def _locnet(xf, w1, b1, w2p, b2p):
    return pl.pallas_call(
        _locnet_body,
        grid=(NSTEPS,),
        in_specs=[
            pl.BlockSpec((B, KC), lambda s: (0, s)),
            pl.BlockSpec((KC, HIDDEN), lambda s: (s, 0)),
            pl.BlockSpec((1, HIDDEN), lambda s: (0, 0)),
            pl.BlockSpec((HIDDEN, 16), lambda s: (0, 0)),
            pl.BlockSpec((1, 16), lambda s: (0, 0)),
        ],
        out_specs=pl.BlockSpec((B, 16), lambda s: (0, 0)),
        out_shape=jax.ShapeDtypeStruct((B, 16), jnp.float32),
        scratch_shapes=[pltpu.VMEM((B, HIDDEN), jnp.float32)],
    )(xf, w1, b1, w2p, b2p)


F32_CHUNK = 8 * W
N_F32_CHUNKS = (H * W) // F32_CHUNK


def _pack_body(ximg_hbm, pk_hbm, img_v, fb0, fb1, sem0, sem1, osem):
    c = lax.axis_index("c")
    s = lax.axis_index("s")
    b = s * 2 + c

    lane = lax.iota(jnp.int32, 16)
    lane2 = lane * 2

    fbufs = (fb0, fb1)
    sems = (sem0, sem1)
    descs = [None, None]
    odescs = [None] * N_F32_CHUNKS

    def start(cc):
        descs[cc % 2] = pltpu.async_copy(
            ximg_hbm.at[b, pl.ds(cc * F32_CHUNK, F32_CHUNK)],
            fbufs[cc % 2], sems[cc % 2])

    def pack_chunk(cc):
        fb = fbufs[cc % 2]

        @plsc.parallel_loop(0, F32_CHUNK // 32, unroll=4)
        def pk(k):
            base = k * 32
            a = plsc.load_gather(fb, [base + lane2])
            bb = plsc.load_gather(fb, [base + lane2 + 1])
            w = plsc.bitcast(
                plsc.pack(a, bb, format=plsc.PackFormat.INTERLEAVED), jnp.int32)
            img_v[pl.ds(cc * (F32_CHUNK // 2) + k * 16, 16)] = w

    start(0)
    for cc in range(N_F32_CHUNKS):
        if cc + 1 < N_F32_CHUNKS:
            start(cc + 1)
        descs[cc % 2].wait()
        pack_chunk(cc)
        odescs[cc] = pltpu.async_copy(
            img_v.at[pl.ds(cc * (F32_CHUNK // 2), F32_CHUNK // 2)],
            pk_hbm.at[b, pl.ds(cc * (F32_CHUNK // 2), F32_CHUNK // 2)], osem)
    for cc in range(N_F32_CHUNKS):
        odescs[cc].wait()


@jax.jit
def _pack(xf):
    mesh = plsc.VectorSubcoreMesh(
        core_axis_name="c", subcore_axis_name="s", num_cores=2, num_subcores=16)
    return pl.kernel(
        _pack_body,
        out_type=jax.ShapeDtypeStruct((B, IMG_WORDS), jnp.int32),
        mesh=mesh,
        compiler_params=pltpu.CompilerParams(needs_layout_passes=False),
        scratch_types=[
            pltpu.VMEM((IMG_WORDS,), jnp.int32),
            pltpu.VMEM((F32_CHUNK,), jnp.float32),
            pltpu.VMEM((F32_CHUNK,), jnp.float32),
            pltpu.SemaphoreType.DMA,
            pltpu.SemaphoreType.DMA,
            pltpu.SemaphoreType.DMA,
        ],
    )(xf)


def _warp_body(pk_hbm, th_hbm, out_hbm, img_v, th_v, jtab, xjt, yjt, ob0, isem):
    c = lax.axis_index("c")
    s = lax.axis_index("s")
    b = s * 2 + c

    idesc = pltpu.async_copy(pk_hbm.at[b], img_v, isem)
    pltpu.sync_copy(th_hbm.at[b], th_v)

    lane = lax.iota(jnp.int32, 16)
    th = th_v[...]

    def bcast(k):
        sel = jnp.where(lane == k, th, jnp.zeros((16,), jnp.float32))
        return jnp.full((16,), jnp.sum(sel, axis=0))

    t00 = bcast(0)
    t01 = bcast(1)
    t02 = bcast(2)
    t10 = bcast(3)
    t11 = bcast(4)
    t12 = bcast(5)

    lanei = lax.iota(jnp.int32, 16)
    lanef = lanei.astype(jnp.float32)
    xmax = jnp.full((16,), W - 1, jnp.float32)
    ymax = jnp.full((16,), H - 1, jnp.float32)
    zero = jnp.zeros((16,), jnp.float32)

    def rbf(v):
        bits = plsc.bitcast(v, jnp.int32)
        r = (bits + 0x7FFF + ((bits >> 16) & 1)) & jnp.int32(-65536)
        return plsc.bitcast(r, jnp.float32)

    @plsc.parallel_loop(0, W // 16, unroll=2)
    def fill_jtab(g):
        jb = rbf(jnp.full((16,), g * 16, jnp.int32).astype(jnp.float32) + lanef)
        jtab[pl.ds(g * 16, 16)] = jb
        xjt[pl.ds(g * 16, 16)] = t00 * jb
        yjt[pl.ds(g * 16, 16)] = t10 * jb

    idesc.wait()

    def do_row(i, rr, obuf):
        iv = rbf(jnp.full((16,), i, jnp.int32).astype(jnp.float32))
        xrow = t01 * iv + t02
        yrow = t11 * iv + t12

        @plsc.parallel_loop(0, W // 16, unroll=4)
        def group(g):
            x = xjt[pl.ds(g * 16, 16)] + xrow
            y = yjt[pl.ds(g * 16, 16)] + yrow
            xlo = jnp.minimum(jnp.maximum(x, zero), xmax)
            xhi = jnp.minimum(jnp.maximum(x + 1.0, zero), xmax)
            ylo = jnp.minimum(jnp.maximum(y, zero), ymax)
            yhi = jnp.minimum(jnp.maximum(y + 1.0, zero), ymax)
            x0 = xlo.astype(jnp.int32)
            x1 = xhi.astype(jnp.int32)
            y0 = ylo.astype(jnp.int32)
            y1 = yhi.astype(jnp.int32)
            x0f = x0.astype(jnp.float32)
            x1f = x1.astype(jnp.float32)
            y0f = y0.astype(jnp.float32)
            y1f = y1.astype(jnp.float32)
            r0 = y0 * WPACK
            r1 = y1 * WPACK
            wxa = x0 >> 1
            wxc = x1 >> 1
            sha = (x0 & 1) << 4
            shc = (x1 & 1) << 4

            def tap(widx, sh):
                wv = plsc.load_gather(img_v, [widx])
                return plsc.bitcast((wv >> sh) << 16, jnp.float32)

            pa = tap(r0 + wxa, sha)
            pc = tap(r0 + wxc, shc)
            pb = tap(r1 + wxa, sha)
            pd = tap(r1 + wxc, shc)

            res = ((y1f - y) * ((x1f - x) * pa + (x - x0f) * pc)
                   + (y - y0f) * ((x1f - x) * pb + (x - x0f) * pd))
            obuf[rr, pl.ds(g * 16, 16)] = res

    def chunk_body(ch, _):
        for rr in range(ROWS_PER_CHUNK):
            do_row(ch * ROWS_PER_CHUNK + rr, rr, ob0)
        pltpu.sync_copy(ob0, out_hbm.at[b, pl.ds(ch * ROWS_PER_CHUNK, ROWS_PER_CHUNK)])
        return 0

    lax.fori_loop(0, H // ROWS_PER_CHUNK, chunk_body, 0)


@jax.jit
def _warp(img, th):
    mesh = plsc.VectorSubcoreMesh(
        core_axis_name="c", subcore_axis_name="s", num_cores=2, num_subcores=16)
    return pl.kernel(
        _warp_body,
        out_type=jax.ShapeDtypeStruct((B, H, W), jnp.float32),
        mesh=mesh,
        compiler_params=pltpu.CompilerParams(needs_layout_passes=False),
        scratch_types=[
            pltpu.VMEM((IMG_WORDS,), jnp.int32),
            pltpu.VMEM((16,), jnp.float32),
            pltpu.VMEM((W,), jnp.float32),
            pltpu.VMEM((W,), jnp.float32),
            pltpu.VMEM((W,), jnp.float32),
            pltpu.VMEM((ROWS_PER_CHUNK, W), jnp.float32),
            pltpu.SemaphoreType.DMA,
        ],
    )(img, th)


def kernel(X, W1, b1, W2, b2):
    xf = X.reshape(B, H * W)
    w2p = jnp.zeros((HIDDEN, 16), jnp.float32).at[:, :6].set(W2)
    b2p = jnp.zeros((16,), jnp.float32).at[:6].set(b2)
    packed = _pack(xf)
    theta16 = _locnet(xf, W1, b1.reshape(1, HIDDEN), w2p, b2p.reshape(1, 16))
    return _warp(packed, theta16)


# speedup vs baseline: 1.3939x; 1.0007x over previous
import functools

import jax
import jax.numpy as jnp
from jax import lax
from jax.experimental import pallas as pl
from jax.experimental.pallas import tpu as pltpu
from jax.experimental.pallas import tpu_sc as plsc

B = 32
H = 384
W = 384
HIDDEN = 128
KC = 4608
NSTEPS = (H * W) // KC
WPACK = W // 2
IMG_WORDS = H * WPACK
ROWS_PER_CHUNK = 8


def _locnet_body(x_ref, w1_ref, b1_ref, w2_ref, b2_ref, th_ref, acc_ref):
    s = pl.program_id(0)

    @pl.when(s == 0)
    def _():
        acc_ref[...] = jnp.zeros_like(acc_ref)

    acc_ref[...] += lax.dot_general(
        x_ref[...], w1_ref[...],
        (((1,), (0,)), ((), ())),
        precision=lax.Precision.DEFAULT,
        preferred_element_type=jnp.float32)

    @pl.when(s == NSTEPS - 1)
    def _():
        h = jnp.maximum(acc_ref[...] + b1_ref[...], 0.0)
        th = lax.dot_general(
            h.astype(jnp.bfloat16), w2_ref[...].astype(jnp.bfloat16),
            (((1,), (0,)), ((), ())),
            preferred_element_type=jnp.float32) + b2_ref[...]
        th_ref[...] = th.astype(jnp.bfloat16).astype(jnp.float32)


@jax.jit
def _locnet(xf, w1, b1, w2p, b2p):
    return pl.pallas_call(
        _locnet_body,
        grid=(NSTEPS,),
        in_specs=[
            pl.BlockSpec((B, KC), lambda s: (0, s)),
            pl.BlockSpec((KC, HIDDEN), lambda s: (s, 0)),
            pl.BlockSpec((1, HIDDEN), lambda s: (0, 0)),
            pl.BlockSpec((HIDDEN, 16), lambda s: (0, 0)),
            pl.BlockSpec((1, 16), lambda s: (0, 0)),
        ],
        out_specs=pl.BlockSpec((B, 16), lambda s: (0, 0)),
        out_shape=jax.ShapeDtypeStruct((B, 16), jnp.float32),
        scratch_shapes=[pltpu.VMEM((B, HIDDEN), jnp.float32)],
    )(xf, w1, b1, w2p, b2p)


F32_CHUNK = 8 * W
N_F32_CHUNKS = (H * W) // F32_CHUNK


def _pack_body(ximg_hbm, pk_hbm, img_v, fb0, fb1, sem0, sem1, osem):
    c = lax.axis_index("c")
    s = lax.axis_index("s")
    b = s * 2 + c

    lane = lax.iota(jnp.int32, 16)
    lane2 = lane * 2

    fbufs = (fb0, fb1)
    sems = (sem0, sem1)
    descs = [None, None]
    odescs = [None] * N_F32_CHUNKS

    def start(cc):
        descs[cc % 2] = pltpu.async_copy(
            ximg_hbm.at[b, pl.ds(cc * F32_CHUNK, F32_CHUNK)],
            fbufs[cc % 2], sems[cc % 2])

    def pack_chunk(cc):
        fb = fbufs[cc % 2]

        @plsc.parallel_loop(0, F32_CHUNK // 32, unroll=4)
        def pk(k):
            base = k * 32
            a = plsc.load_gather(fb, [base + lane2])
            bb = plsc.load_gather(fb, [base + lane2 + 1])
            w = plsc.bitcast(
                plsc.pack(a, bb, format=plsc.PackFormat.INTERLEAVED), jnp.int32)
            img_v[pl.ds(cc * (F32_CHUNK // 2) + k * 16, 16)] = w

    start(0)
    for cc in range(N_F32_CHUNKS):
        if cc + 1 < N_F32_CHUNKS:
            start(cc + 1)
        descs[cc % 2].wait()
        pack_chunk(cc)
        odescs[cc] = pltpu.async_copy(
            img_v.at[pl.ds(cc * (F32_CHUNK // 2), F32_CHUNK // 2)],
            pk_hbm.at[b, pl.ds(cc * (F32_CHUNK // 2), F32_CHUNK // 2)], osem)
    for cc in range(N_F32_CHUNKS):
        odescs[cc].wait()


@jax.jit
def _pack(xf):
    mesh = plsc.VectorSubcoreMesh(
        core_axis_name="c", subcore_axis_name="s", num_cores=2, num_subcores=16)
    return pl.kernel(
        _pack_body,
        out_type=jax.ShapeDtypeStruct((B, IMG_WORDS), jnp.int32),
        mesh=mesh,
        compiler_params=pltpu.CompilerParams(needs_layout_passes=False),
        scratch_types=[
            pltpu.VMEM((IMG_WORDS,), jnp.int32),
            pltpu.VMEM((F32_CHUNK,), jnp.float32),
            pltpu.VMEM((F32_CHUNK,), jnp.float32),
            pltpu.SemaphoreType.DMA,
            pltpu.SemaphoreType.DMA,
            pltpu.SemaphoreType.DMA,
        ],
    )(xf)


def _warp_body(pk_hbm, th_hbm, out_hbm, img_v, th_v, jtab, xjt, yjt, ob0, isem):
    c = lax.axis_index("c")
    s = lax.axis_index("s")
    b = s * 2 + c

    idesc = pltpu.async_copy(pk_hbm.at[b], img_v, isem)
    pltpu.sync_copy(th_hbm.at[b], th_v)

    lane = lax.iota(jnp.int32, 16)
    th = th_v[...]

    def bcast(k):
        sel = jnp.where(lane == k, th, jnp.zeros((16,), jnp.float32))
        return jnp.full((16,), jnp.sum(sel, axis=0))

    t00 = bcast(0)
    t01 = bcast(1)
    t02 = bcast(2)
    t10 = bcast(3)
    t11 = bcast(4)
    t12 = bcast(5)

    lanei = lax.iota(jnp.int32, 16)
    lanef = lanei.astype(jnp.float32)
    xmax = jnp.full((16,), W - 1, jnp.float32)
    ymax = jnp.full((16,), H - 1, jnp.float32)
    zero = jnp.zeros((16,), jnp.float32)

    def rbf(v):
        bits = plsc.bitcast(v, jnp.int32)
        r = (bits + 0x7FFF + ((bits >> 16) & 1)) & jnp.int32(-65536)
        return plsc.bitcast(r, jnp.float32)

    @plsc.parallel_loop(0, W // 16, unroll=2)
    def fill_jtab(g):
        jb = rbf(jnp.full((16,), g * 16, jnp.int32).astype(jnp.float32) + lanef)
        jtab[pl.ds(g * 16, 16)] = jb
        xjt[pl.ds(g * 16, 16)] = t00 * jb
        yjt[pl.ds(g * 16, 16)] = t10 * jb

    idesc.wait()

    def do_row(i, rr, obuf):
        iv = rbf(jnp.full((16,), i, jnp.int32).astype(jnp.float32))
        xrow = t01 * iv + t02
        yrow = t11 * iv + t12

        @plsc.parallel_loop(0, W // 16, unroll=4)
        def group(g):
            x = xjt[pl.ds(g * 16, 16)] + xrow
            y = yjt[pl.ds(g * 16, 16)] + yrow
            xlo = jnp.minimum(jnp.maximum(x, zero), xmax)
            xhi = jnp.minimum(jnp.maximum(x + 1.0, zero), xmax)
            ylo = jnp.minimum(jnp.maximum(y, zero), ymax)
            yhi = jnp.minimum(jnp.maximum(y + 1.0, zero), ymax)
            x0 = xlo.astype(jnp.int32)
            x1 = xhi.astype(jnp.int32)
            y0 = ylo.astype(jnp.int32)
            y1 = yhi.astype(jnp.int32)
            x0f = x0.astype(jnp.float32)
            x1f = x1.astype(jnp.float32)
            y0f = y0.astype(jnp.float32)
            y1f = y1.astype(jnp.float32)
            r0 = y0 * WPACK
            r1 = y1 * WPACK
            wxa = x0 >> 1
            wxc = x1 >> 1
            sha = (x0 & 1) << 4
            shc = (x1 & 1) << 4

            def tap(widx, sh):
                wv = plsc.load_gather(img_v, [widx])
                return plsc.bitcast((wv >> sh) << 16, jnp.float32)

            pa = tap(r0 + wxa, sha)
            pc = tap(r0 + wxc, shc)
            pb = tap(r1 + wxa, sha)
            pd = tap(r1 + wxc, shc)

            res = ((y1f - y) * ((x1f - x) * pa + (x - x0f) * pc)
                   + (y - y0f) * ((x1f - x) * pb + (x - x0f) * pd))
            obuf[rr, pl.ds(g * 16, 16)] = res

    def chunk_body(ch, _):
        for rr in range(ROWS_PER_CHUNK):
            do_row(ch * ROWS_PER_CHUNK + rr, rr, ob0)
        pltpu.sync_copy(ob0, out_hbm.at[b, pl.ds(ch * ROWS_PER_CHUNK, ROWS_PER_CHUNK)])
        return 0

    lax.fori_loop(0, H // ROWS_PER_CHUNK, chunk_body, 0)


@jax.jit
def _warp(img, th):
    mesh = plsc.VectorSubcoreMesh(
        core_axis_name="c", subcore_axis_name="s", num_cores=2, num_subcores=16)
    return pl.kernel(
        _warp_body,
        out_type=jax.ShapeDtypeStruct((B, H, W), jnp.float32),
        mesh=mesh,
        compiler_params=pltpu.CompilerParams(needs_layout_passes=False),
        scratch_types=[
            pltpu.VMEM((IMG_WORDS,), jnp.int32),
            pltpu.VMEM((16,), jnp.float32),
            pltpu.VMEM((W,), jnp.float32),
            pltpu.VMEM((W,), jnp.float32),
            pltpu.VMEM((W,), jnp.float32),
            pltpu.VMEM((ROWS_PER_CHUNK, W), jnp.float32),
            pltpu.SemaphoreType.DMA,
        ],
    )(img, th)


def kernel(X, W1, b1, W2, b2):
    xf = X.reshape(B, H * W)
    w2p = jnp.zeros((HIDDEN, 16), jnp.float32).at[:, :6].set(W2)
    b2p = jnp.zeros((16,), jnp.float32).at[:6].set(b2)
    packed = _pack(xf)
    theta16 = _locnet(xf, W1, b1.reshape(1, HIDDEN), w2p, b2p.reshape(1, 16))
    return _warp(packed, theta16)
